# Initial kernel scaffold; baseline (speedup 1.0000x reference)
import jax
import jax.numpy as jnp
from jax.experimental import pallas as pl


def kernel(x, W_E):
    raise NotImplementedError("write your pallas kernel here")


# speedup vs baseline: 1.5218x; 1.5218x over previous
import functools

import jax
import jax.numpy as jnp
from jax import lax
from jax.experimental import pallas as pl
from jax.experimental.pallas import tpu as pltpu
from jax.experimental.pallas import tpu_sc as plsc

NC = 2
NS = 16
NW = NC * NS
CHUNK = 128
NBUF = 4


@functools.partial(jax.jit, static_argnames=("n_chunks", "d_embed"))
def _embed_sc(x_grp, w, n_chunks, d_embed):
    n_total = NW * n_chunks * CHUNK
    mesh = plsc.VectorSubcoreMesh(core_axis_name="c", subcore_axis_name="s")

    def body(x_hbm, w_hbm, out_hbm, idx_v, rows_v, gsems):
        wid = lax.axis_index("s") * NC + lax.axis_index("c")
        base = wid * (n_chunks * CHUNK)
        pltpu.sync_copy(x_hbm.at[wid], idx_v)
        for j in range(min(NBUF, n_chunks)):
            pltpu.async_copy(w_hbm.at[idx_v.at[j]], rows_v.at[j], gsems.at[j])
        for j in range(n_chunks):
            b = j % NBUF
            pltpu.make_async_copy(
                w_hbm.at[idx_v.at[j]], rows_v.at[b], gsems.at[b]).wait()
            pltpu.sync_copy(
                rows_v.at[b], out_hbm.at[pl.ds(base + j * CHUNK, CHUNK)])
            nj = j + NBUF
            if nj < n_chunks:
                pltpu.async_copy(w_hbm.at[idx_v.at[nj]], rows_v.at[b],
                                 gsems.at[b])

    run = pl.kernel(
        body,
        out_type=jax.ShapeDtypeStruct((n_total, d_embed), w.dtype),
        mesh=mesh,
        scratch_types=[
            pltpu.VMEM((n_chunks, CHUNK), jnp.int32),
            pltpu.VMEM((NBUF, CHUNK, d_embed), w.dtype),
            pltpu.SemaphoreType.DMA((NBUF,)),
        ],
    )
    return run(x_grp, w)


def kernel(x, W_E):
    n_total = x.size
    d_embed = W_E.shape[1]
    assert n_total % (NW * CHUNK) == 0
    n_chunks = n_total // (NW * CHUNK)
    x_grp = x.reshape(NW, n_chunks, CHUNK).astype(jnp.int32)
    out = _embed_sc(x_grp, W_E, n_chunks, d_embed)
    return out.reshape(x.shape + (d_embed,))


# speedup vs baseline: 1.5629x; 1.0270x over previous
import functools

import jax
import jax.numpy as jnp
from jax import lax
from jax.experimental import pallas as pl
from jax.experimental.pallas import tpu as pltpu
from jax.experimental.pallas import tpu_sc as plsc

NC = 2
NS = 16
NW = NC * NS
CHUNK = 128
NBUF = 6


@functools.partial(jax.jit, static_argnames=("n_chunks", "d_embed"))
def _embed_sc(x_grp, w, n_chunks, d_embed):
    n_total = NW * n_chunks * CHUNK
    mesh = plsc.VectorSubcoreMesh(core_axis_name="c", subcore_axis_name="s")

    def body(x_hbm, w_hbm, out_hbm, idx_v, rows_v, gsems, wsems):
        wid = lax.axis_index("s") * NC + lax.axis_index("c")
        base = wid * (n_chunks * CHUNK)
        pltpu.sync_copy(x_hbm.at[wid], idx_v)

        def out_at(j):
            return out_hbm.at[pl.ds(base + j * CHUNK, CHUNK)]

        for j in range(min(NBUF, n_chunks)):
            pltpu.async_copy(w_hbm.at[idx_v.at[j]], rows_v.at[j], gsems.at[j])
        for j in range(n_chunks):
            b = j % NBUF
            pltpu.make_async_copy(
                w_hbm.at[idx_v.at[j]], rows_v.at[b], gsems.at[b]).wait()
            pltpu.async_copy(rows_v.at[b], out_at(j), wsems.at[b])
            nj = j + NBUF
            if nj < n_chunks:
                pltpu.make_async_copy(rows_v.at[b], out_at(j),
                                      wsems.at[b]).wait()
                pltpu.async_copy(w_hbm.at[idx_v.at[nj]], rows_v.at[b],
                                 gsems.at[b])
        for j in range(max(0, n_chunks - NBUF), n_chunks):
            b = j % NBUF
            pltpu.make_async_copy(rows_v.at[b], out_at(j), wsems.at[b]).wait()

    run = pl.kernel(
        body,
        out_type=jax.ShapeDtypeStruct((n_total, d_embed), w.dtype),
        mesh=mesh,
        scratch_types=[
            pltpu.VMEM((n_chunks, CHUNK), jnp.int32),
            pltpu.VMEM((NBUF, CHUNK, d_embed), w.dtype),
            pltpu.SemaphoreType.DMA((NBUF,)),
            pltpu.SemaphoreType.DMA((NBUF,)),
        ],
    )
    return run(x_grp, w)


def kernel(x, W_E):
    n_total = x.size
    d_embed = W_E.shape[1]
    assert n_total % (NW * CHUNK) == 0
    n_chunks = n_total // (NW * CHUNK)
    x_grp = x.reshape(NW, n_chunks, CHUNK).astype(jnp.int32)
    out = _embed_sc(x_grp, W_E, n_chunks, d_embed)
    return out.reshape(x.shape + (d_embed,))
